# baseline (speedup 1.0000x reference)
import functools
import jax
import jax.numpy as jnp
from jax import lax
from jax.experimental import pallas as pl
from jax.experimental.pallas import tpu as pltpu
from jax.experimental.pallas import tpu_sc as plsc

_NC, _NS, _LANES = 2, 16, 16
_NW = _NC * _NS
_D = 128
_T = 64
_UNROLL = 4

_GDN = lax.GatherDimensionNumbers(offset_dims=(), collapsed_slice_dims=(0,),
                                  start_index_map=(0,))


def _bcast_lane(v, lane):
    gi = jnp.full((_LANES, 1), lane, jnp.int32)
    return lax.gather(v, gi, _GDN, (1,),
                      mode=lax.GatherScatterMode.PROMISE_IN_BOUNDS)


def _sc_body(coords_hbm, at_hbm, rt_hbm, mt_hbm, W_hbm, b_hbm,
             atab_hbm, rtab_hbm, mtab_hbm, out_hbm,
             idxa_v, idxr_v, idxm_v, coords_v,
             ra0, rr0, rm0, ra1, rr1, rm1, out0, out1, W_v, b_v,
             sa0, sr0, sm0, sa1, sr1, sm1, so0, so1, n_tok):
    pw = n_tok // _NW
    nch = pw // _T
    wid = lax.axis_index("s") * _NC + lax.axis_index("c")
    base = wid * pw

    rows = [(ra0, rr0, rm0), (ra1, rr1, rm1)]
    gsems = [(sa0, sr0, sm0), (sa1, sr1, sm1)]
    outs = [out0, out1]
    osems = [so0, so1]
    tabs = (atab_hbm, rtab_hbm, mtab_hbm)
    idxs = (idxa_v, idxr_v, idxm_v)

    pltpu.sync_copy(W_hbm, W_v)
    pltpu.sync_copy(b_hbm, b_v)
    pltpu.sync_copy(at_hbm.at[pl.ds(wid * nch, nch)], idxa_v)
    pltpu.sync_copy(rt_hbm.at[pl.ds(wid * nch, nch)], idxr_v)
    pltpu.sync_copy(mt_hbm.at[pl.ds(wid * nch, nch)], idxm_v)
    pltpu.sync_copy(coords_hbm.at[pl.ds(base * 3, pw * 3)],
                    coords_v.at[pl.ds(0, pw * 3)])
    Wc = [[W_v[pl.ds(c * _D + 16 * k, 16)] for k in range(8)] for c in range(3)]
    bc = [b_v[pl.ds(16 * k, 16)] for k in range(8)]

    def issue_gathers(buf, ci):
        for tab, ix, rbuf, sem in zip(tabs, idxs, rows[buf], gsems[buf]):
            pltpu.async_copy(tab.at[ix.at[ci]], rbuf, sem)

    issue_gathers(0, 0)
    issue_gathers(1, 1)

    def pair_body(cp, carry):
        for b2 in (0, 1):
            ci = cp * 2 + b2
            cb = base + ci * _T
            for tab, ix, rbuf, sem in zip(tabs, idxs, rows[b2], gsems[b2]):
                pltpu.make_async_copy(tab.at[ix.at[ci]], rbuf, sem).wait()

            @pl.when(cp > 0)
            def _wait_out():
                pltpu.make_async_copy(outs[b2],
                                      out_hbm.at[pl.ds(cb, _T)],
                                      osems[b2]).wait()

            ra, rr, rm = rows[b2]
            ov = outs[b2]
            cbase3 = ci * (_T * 3)

            @plsc.parallel_loop(0, _T, 1, unroll=_UNROLL)
            def tok_body(t):
                v = coords_v[pl.ds(cbase3 + 3 * t, _LANES)]
                bx = _bcast_lane(v, 0)
                by = _bcast_lane(v, 1)
                bz = _bcast_lane(v, 2)
                for k in range(8):
                    sl = pl.ds(16 * k, 16)
                    pr = bc[k] + bx * Wc[0][k] + by * Wc[1][k] + bz * Wc[2][k]
                    h = pr / (1.0 + jnp.exp(-pr))
                    vv = h + ra[t, sl] + rr[t, sl] + rm[t, sl]
                    ov[t, sl] = vv

            pltpu.async_copy(ov, out_hbm.at[pl.ds(cb, _T)], osems[b2])

            @pl.when(ci + 2 < nch)
            def _next_gathers():
                issue_gathers(b2, ci + 2)
        return carry

    lax.fori_loop(0, nch // 2, pair_body, 0)

    for b2 in (0, 1):
        cb = base + (nch - 2 + b2) * _T
        pltpu.make_async_copy(outs[b2], out_hbm.at[pl.ds(cb, _T)],
                              osems[b2]).wait()


def kernel(coords, atom_types, residue_types, meta_classes, W_coord, b_coord,
           atom_table, residue_table, meta_table):
    B, L, D = coords.shape[0], coords.shape[1], W_coord.shape[1]
    N = B * L
    pw = N // _NW
    nch = pw // _T
    coords_f = coords.reshape(N * 3)
    at = atom_types.reshape(_NW * nch, _T)
    rt = residue_types.reshape(_NW * nch, _T)
    mt = meta_classes.reshape(_NW * nch, _T)
    W_f = W_coord.reshape(3 * D)

    mesh = plsc.VectorSubcoreMesh(core_axis_name="c", subcore_axis_name="s",
                                  num_cores=_NC, num_subcores=_NS)
    sc_fn = pl.kernel(
        functools.partial(_sc_body, n_tok=N),
        out_type=jax.ShapeDtypeStruct((N, _D), jnp.float32),
        mesh=mesh,
        scratch_types=[
            pltpu.VMEM((nch, _T), jnp.int32),
            pltpu.VMEM((nch, _T), jnp.int32),
            pltpu.VMEM((nch, _T), jnp.int32),
            pltpu.VMEM((pw * 3 + _LANES,), jnp.float32),
            pltpu.VMEM((_T, _D), jnp.float32),
            pltpu.VMEM((_T, _D), jnp.float32),
            pltpu.VMEM((_T, _D), jnp.float32),
            pltpu.VMEM((_T, _D), jnp.float32),
            pltpu.VMEM((_T, _D), jnp.float32),
            pltpu.VMEM((_T, _D), jnp.float32),
            pltpu.VMEM((_T, _D), jnp.float32),
            pltpu.VMEM((_T, _D), jnp.float32),
            pltpu.VMEM((3 * _D,), jnp.float32),
            pltpu.VMEM((_D,), jnp.float32),
        ] + [pltpu.SemaphoreType.DMA] * 8,
    )
    out = sc_fn(coords_f, at, rt, mt, W_f, b_coord,
                atom_table, residue_table, meta_table)
    return out.reshape(B, L, D)
